# baseline (speedup 1.0000x reference)
import jax
import jax.numpy as jnp
from jax import lax
from jax.experimental import pallas as pl
from jax.experimental.pallas import tpu as pltpu
from jax.experimental.pallas import tpu_sc as plsc

N, P, C = 10000, 32, 128
NW = 32
NPAD = 10240
VPW = NPAD // NW
VB = 8
CHUNKS = VPW // VB
EPC = VB * P
IDX_B = 128
NIDX = EPC // IDX_B
L = 16
CV = C // L

_SPLAT_DNUMS = lax.GatherDimensionNumbers(
    offset_dims=(), collapsed_slice_dims=(0,), start_index_map=(0,)
)


def _splat(vec, elt):
    return lax.gather(
        vec,
        jnp.full((L, 1), elt, jnp.int32),
        _SPLAT_DNUMS,
        slice_sizes=(1,),
        mode=lax.GatherScatterMode.PROMISE_IN_BOUNDS,
    )


def _sc_kernel(pairs_hbm, w0_hbm, w1_hbm, table_hbm, out_hbm,
               pairs_v, idx_v, w0_v, w1_v, rows_v, out_v, sem0, sem1):
    wid = lax.axis_index("s") * 2 + lax.axis_index("c")
    vbase0 = pl.multiple_of(jnp.minimum(wid * VPW, N - VPW), 16)
    pltpu.sync_copy(
        pairs_hbm.at[pl.ds(pl.multiple_of(vbase0 * P * 2, 512), VPW * P * 2)],
        pairs_v)
    lane = lax.iota(jnp.int32, L)
    odd = ((lane & 7) * 2 + 1).reshape(L, 1)
    lane_lo = lane < 8

    def _permute(vec):
        return lax.gather(vec, odd, _SPLAT_DNUMS, slice_sizes=(1,),
                          mode=lax.GatherScatterMode.PROMISE_IN_BOUNDS)

    def extract_body(i, carry):
        base = i * (2 * L)
        a = pairs_v[pl.ds(base, L)]
        bvec = pairs_v[pl.ds(base + L, L)]
        idx_v[pl.ds(i * L, L)] = jnp.where(lane_lo, _permute(a), _permute(bvec))
        return carry

    lax.fori_loop(0, VPW * P // L, extract_body, 0)
    sems = (sem0, sem1)

    def issue(g, b):
        ebase = pl.multiple_of((vbase0 + g * VB) * P, 256)
        sem = sems[b]
        for j in range(NIDX):
            pltpu.async_copy(
                table_hbm.at[idx_v.at[pl.ds((g * NIDX + j) * IDX_B, IDX_B)]],
                rows_v.at[pl.ds((b * NIDX + j) * IDX_B, IDX_B)],
                sem,
            )
        pltpu.async_copy(w0_hbm.at[pl.ds(ebase, EPC)],
                         w0_v.at[pl.ds(b * EPC, EPC)], sem)
        pltpu.async_copy(w1_hbm.at[pl.ds(ebase * 3, EPC * 3)],
                         w1_v.at[pl.ds(b * EPC * 3, EPC * 3)], sem)

    def wait(g, b):
        ebase = pl.multiple_of((vbase0 + g * VB) * P, 256)
        sem = sems[b]
        for j in range(NIDX):
            pltpu.make_async_copy(
                table_hbm.at[idx_v.at[pl.ds((g * NIDX + j) * IDX_B, IDX_B)]],
                rows_v.at[pl.ds((b * NIDX + j) * IDX_B, IDX_B)],
                sem,
            ).wait()
        pltpu.make_async_copy(w0_hbm.at[pl.ds(ebase, EPC)],
                              w0_v.at[pl.ds(b * EPC, EPC)], sem).wait()
        pltpu.make_async_copy(w1_hbm.at[pl.ds(ebase * 3, EPC * 3)],
                              w1_v.at[pl.ds(b * EPC * 3, EPC * 3)], sem).wait()

    def compute(g, b):
        def v_body(v, carry):
            accs = [jnp.zeros((L,), jnp.float32) for _ in range(4 * CV)]
            row_base = b * EPC + v * P
            w0base = b * EPC + v * P
            w1base = (b * EPC + v * P) * 3
            w0vecs = [w0_v[pl.ds(w0base + j * L, L)] for j in range(P // L)]
            w1vecs = [w1_v[pl.ds(w1base + j * L, L)] for j in range(P * 3 // L)]
            for p in range(P):
                ws = [_splat(w0vecs[p // L], p % L)]
                for k in range(3):
                    off = p * 3 + k
                    ws.append(_splat(w1vecs[off // L], off % L))
                for c8 in range(CV):
                    row = rows_v[row_base + p, pl.ds(c8 * L, L)]
                    for k in range(4):
                        accs[k * CV + c8] = accs[k * CV + c8] + ws[k] * row
            for k in range(4):
                for c8 in range(CV):
                    out_v[v * 4 + k, pl.ds(c8 * L, L)] = accs[k * CV + c8]
            return carry

        lax.fori_loop(0, VB, v_body, 0)
        pltpu.sync_copy(
            out_v,
            out_hbm.at[pl.ds(pl.multiple_of((vbase0 + g * VB) * 4, 32), VB * 4)],
        )

    issue(0, 0)

    def loop_body(gg, carry):
        g0 = gg * 2
        wait(g0, 0)
        issue(g0 + 1, 1)
        compute(g0, 0)
        wait(g0 + 1, 1)

        @pl.when(g0 + 2 < CHUNKS)
        def _():
            issue(g0 + 2, 0)

        compute(g0 + 1, 1)
        return carry

    lax.fori_loop(0, CHUNKS // 2, loop_body, 0)


@jax.jit
def kernel(signal_0, kernel_0, kernel_1, patches_idx):
    table = signal_0[0, :, 0, :]
    w0 = kernel_0.reshape(N * P)
    w1 = kernel_1.reshape(N * P * 3)
    pairs = patches_idx.reshape(N * P * 2)

    mesh = plsc.VectorSubcoreMesh(core_axis_name="c", subcore_axis_name="s")
    out = pl.kernel(
        _sc_kernel,
        out_type=jax.ShapeDtypeStruct((N * 4, C), jnp.float32),
        mesh=mesh,
        scratch_types=[
            pltpu.VMEM((VPW * P * 2,), jnp.int32),
            pltpu.VMEM((VPW * P,), jnp.int32),
            pltpu.VMEM((2 * EPC,), jnp.float32),
            pltpu.VMEM((2 * EPC * 3,), jnp.float32),
            pltpu.VMEM((2 * EPC, C), jnp.float32),
            pltpu.VMEM((VB * 4, C), jnp.float32),
            pltpu.SemaphoreType.DMA,
            pltpu.SemaphoreType.DMA,
        ],
    )(pairs, w0, w1, table)

    y = out.reshape(N, 4, C)
    y0 = y[None, :, 0:1, :]
    y1 = y[None, :, 1:4, :]
    return (y0, y1)


# speedup vs baseline: 1.4003x; 1.4003x over previous
import jax
import jax.numpy as jnp
from jax import lax
from jax.experimental import pallas as pl
from jax.experimental.pallas import tpu as pltpu
from jax.experimental.pallas import tpu_sc as plsc

N, P, C = 10000, 32, 128
NW = 32
NPAD = 10240
VPW = NPAD // NW
VB = 8
CHUNKS = VPW // VB
EPC = VB * P
IDX_B = 128
NIDX = EPC // IDX_B
L = 16
CV = C // L

_SPLAT_DNUMS = lax.GatherDimensionNumbers(
    offset_dims=(), collapsed_slice_dims=(0,), start_index_map=(0,)
)


def _splat(vec, elt):
    return lax.gather(
        vec,
        jnp.full((L, 1), elt, jnp.int32),
        _SPLAT_DNUMS,
        slice_sizes=(1,),
        mode=lax.GatherScatterMode.PROMISE_IN_BOUNDS,
    )


def _sc_kernel(idx_hbm, w0_hbm, w1_hbm, table_hbm, out0_hbm, out1_hbm,
               idx_v, w0_v, w1_v, rows_v, out0_v, out1_v, sem0, sem1):
    wid = lax.axis_index("s") * 2 + lax.axis_index("c")
    vbase0 = pl.multiple_of(jnp.minimum(wid * VPW, N - VPW), 16)
    pltpu.sync_copy(
        idx_hbm.at[pl.ds(pl.multiple_of(vbase0 * P, 256), VPW * P)],
        idx_v)
    sems = (sem0, sem1)

    def issue(g, b):
        ebase = pl.multiple_of((vbase0 + g * VB) * P, 256)
        sem = sems[b]
        for j in range(NIDX):
            pltpu.async_copy(
                table_hbm.at[idx_v.at[pl.ds((g * NIDX + j) * IDX_B, IDX_B)]],
                rows_v.at[pl.ds((b * NIDX + j) * IDX_B, IDX_B)],
                sem,
            )
        pltpu.async_copy(w0_hbm.at[pl.ds(ebase, EPC)],
                         w0_v.at[pl.ds(b * EPC, EPC)], sem)
        pltpu.async_copy(w1_hbm.at[pl.ds(ebase * 3, EPC * 3)],
                         w1_v.at[pl.ds(b * EPC * 3, EPC * 3)], sem)

    def wait(g, b):
        ebase = pl.multiple_of((vbase0 + g * VB) * P, 256)
        sem = sems[b]
        for j in range(NIDX):
            pltpu.make_async_copy(
                table_hbm.at[idx_v.at[pl.ds((g * NIDX + j) * IDX_B, IDX_B)]],
                rows_v.at[pl.ds((b * NIDX + j) * IDX_B, IDX_B)],
                sem,
            ).wait()
        pltpu.make_async_copy(w0_hbm.at[pl.ds(ebase, EPC)],
                              w0_v.at[pl.ds(b * EPC, EPC)], sem).wait()
        pltpu.make_async_copy(w1_hbm.at[pl.ds(ebase * 3, EPC * 3)],
                              w1_v.at[pl.ds(b * EPC * 3, EPC * 3)], sem).wait()

    def compute(g, b):
        def v_body(v, carry):
            accs = [jnp.zeros((L,), jnp.float32) for _ in range(4 * CV)]
            row_base = b * EPC + v * P
            w0base = b * EPC + v * P
            w1base = (b * EPC + v * P) * 3
            w0vecs = [w0_v[pl.ds(w0base + j * L, L)] for j in range(P // L)]
            w1vecs = [w1_v[pl.ds(w1base + j * L, L)] for j in range(P * 3 // L)]
            for p in range(P):
                ws = [_splat(w0vecs[p // L], p % L)]
                for k in range(3):
                    off = p * 3 + k
                    ws.append(_splat(w1vecs[off // L], off % L))
                for c8 in range(CV):
                    row = rows_v[row_base + p, pl.ds(c8 * L, L)]
                    for k in range(4):
                        accs[k * CV + c8] = accs[k * CV + c8] + ws[k] * row
            for c8 in range(CV):
                out0_v[v, pl.ds(c8 * L, L)] = accs[c8]
            for k in range(1, 4):
                for c8 in range(CV):
                    out1_v[v * 3 + (k - 1), pl.ds(c8 * L, L)] = accs[k * CV + c8]
            return carry

        lax.fori_loop(0, VB, v_body, 0)
        vb = vbase0 + g * VB
        pltpu.sync_copy(
            out0_v, out0_hbm.at[pl.ds(pl.multiple_of(vb, 8), VB)])
        pltpu.sync_copy(
            out1_v, out1_hbm.at[pl.ds(pl.multiple_of(vb * 3, 24), VB * 3)])

    issue(0, 0)

    def loop_body(gg, carry):
        g0 = gg * 2
        wait(g0, 0)
        issue(g0 + 1, 1)
        compute(g0, 0)
        wait(g0 + 1, 1)

        @pl.when(g0 + 2 < CHUNKS)
        def _():
            issue(g0 + 2, 0)

        compute(g0 + 1, 1)
        return carry

    lax.fori_loop(0, CHUNKS // 2, loop_body, 0)


@jax.jit
def kernel(signal_0, kernel_0, kernel_1, patches_idx):
    table = signal_0[0, :, 0, :]
    w0 = kernel_0.reshape(N * P)
    w1 = kernel_1.reshape(N * P * 3)
    idx = patches_idx[0, :, :, 1].reshape(N * P)

    mesh = plsc.VectorSubcoreMesh(core_axis_name="c", subcore_axis_name="s")
    out0, out1 = pl.kernel(
        _sc_kernel,
        out_type=(
            jax.ShapeDtypeStruct((N, C), jnp.float32),
            jax.ShapeDtypeStruct((N * 3, C), jnp.float32),
        ),
        mesh=mesh,
        scratch_types=[
            pltpu.VMEM((VPW * P,), jnp.int32),
            pltpu.VMEM((2 * EPC,), jnp.float32),
            pltpu.VMEM((2 * EPC * 3,), jnp.float32),
            pltpu.VMEM((2 * EPC, C), jnp.float32),
            pltpu.VMEM((VB, C), jnp.float32),
            pltpu.VMEM((VB * 3, C), jnp.float32),
            pltpu.SemaphoreType.DMA,
            pltpu.SemaphoreType.DMA,
        ],
    )(idx, w0, w1, table)

    y0 = out0.reshape(1, N, 1, C)
    y1 = out1.reshape(1, N, 3, C)
    return (y0, y1)


# speedup vs baseline: 2.7783x; 1.9840x over previous
import jax
import jax.numpy as jnp
from jax import lax
from jax.experimental import pallas as pl
from jax.experimental.pallas import tpu as pltpu
from jax.experimental.pallas import tpu_sc as plsc

N, P, C = 10000, 32, 128
NW = 32
NPAD = 10240
VPW = NPAD // NW
VB = 8
CHUNKS = VPW // VB
EPC = VB * P
IDX_B = 128
L = 16
CV = C // L
NV_E = EPC // L
NV_W1 = 3 * EPC // L

_SPLAT_DNUMS = lax.GatherDimensionNumbers(
    offset_dims=(), collapsed_slice_dims=(0,), start_index_map=(0,)
)


def _splat(vec, elt):
    return lax.gather(
        vec,
        jnp.full((L, 1), elt, jnp.int32),
        _SPLAT_DNUMS,
        slice_sizes=(1,),
        mode=lax.GatherScatterMode.PROMISE_IN_BOUNDS,
    )


def _sc_kernel(idx_hbm, w0_hbm, w1_hbm, table_hbm, out0_hbm, out1_hbm,
               pat_idx, pat_w0, pat_w1, lst_idx, lst_w0, lst_w1,
               idxval, w0_v, w1_v, rows_v, out0_v, out1_v,
               sem_i0, sem_i1, sem_r0, sem_r1):
    wid = lax.axis_index("s") * 2 + lax.axis_index("c")
    vbase0 = pl.multiple_of(jnp.minimum(wid * VPW, N - VPW), 16)
    lane = lax.iota(jnp.int32, L)
    sem_i = (sem_i0, sem_i1)
    sem_r = (sem_r0, sem_r1)

    def pat_edge_body(i, carry):
        m = i * L + lane
        v = m >> 5
        p = m & (P - 1)
        pat_idx[i, :] = p * (2 * N) + N + v
        pat_w0[i, :] = p * N + v
        return carry

    lax.fori_loop(0, NV_E, pat_edge_body, 0)

    def pat_w1_body(v, carry):
        for r in range(6):
            q = r * L + lane
            k = q >> 5
            p = q & (P - 1)
            pat_w1[v * 6 + r, :] = (p * 3 + k) * N + v
        return carry

    lax.fori_loop(0, VB, pat_w1_body, 0)

    def issue_idx(g, s):
        vv = jnp.full((L,), vbase0 + g * VB, jnp.int32)
        for i in range(NV_E):
            lst_idx[s * 2 + i // 8, pl.ds((i % 8) * L, L)] = pat_idx[i, :] + vv
        for j in range(2):
            pltpu.async_copy(idx_hbm.at[lst_idx.at[s * 2 + j]],
                             idxval.at[s * 2 + j], sem_i[s])

    def wait_idx(s):
        for j in range(2):
            pltpu.make_async_copy(idx_hbm.at[lst_idx.at[s * 2 + j]],
                                  idxval.at[s * 2 + j], sem_i[s]).wait()

    def issue_rows_w(g, s):
        vv = jnp.full((L,), vbase0 + g * VB, jnp.int32)
        for j in range(2):
            pltpu.async_copy(
                table_hbm.at[idxval.at[s * 2 + j]],
                rows_v.at[pl.ds((s * 2 + j) * IDX_B, IDX_B)], sem_r[s])
        for i in range(NV_E):
            lst_w0[s * 2 + i // 8, pl.ds((i % 8) * L, L)] = pat_w0[i, :] + vv
        for i in range(NV_W1):
            lst_w1[s * 6 + i // 8, pl.ds((i % 8) * L, L)] = pat_w1[i, :] + vv
        for j in range(2):
            pltpu.async_copy(w0_hbm.at[lst_w0.at[s * 2 + j]],
                             w0_v.at[pl.ds((s * 2 + j) * IDX_B, IDX_B)],
                             sem_r[s])
        for j in range(6):
            pltpu.async_copy(w1_hbm.at[lst_w1.at[s * 6 + j]],
                             w1_v.at[pl.ds((s * 6 + j) * IDX_B, IDX_B)],
                             sem_r[s])

    def wait_rows_w(s):
        for j in range(2):
            pltpu.make_async_copy(
                table_hbm.at[idxval.at[s * 2 + j]],
                rows_v.at[pl.ds((s * 2 + j) * IDX_B, IDX_B)], sem_r[s]).wait()
        for j in range(2):
            pltpu.make_async_copy(
                w0_hbm.at[lst_w0.at[s * 2 + j]],
                w0_v.at[pl.ds((s * 2 + j) * IDX_B, IDX_B)], sem_r[s]).wait()
        for j in range(6):
            pltpu.make_async_copy(
                w1_hbm.at[lst_w1.at[s * 6 + j]],
                w1_v.at[pl.ds((s * 6 + j) * IDX_B, IDX_B)], sem_r[s]).wait()

    def compute(g, b):
        def v_body(v, carry):
            accs = [jnp.zeros((L,), jnp.float32) for _ in range(4 * CV)]
            row_base = b * EPC + v * P
            w0b = b * EPC + v * P
            w1b = b * EPC * 3 + v * (3 * P)
            w0vecs = [w0_v[pl.ds(w0b + j * L, L)] for j in range(P // L)]
            w1vecs = [w1_v[pl.ds(w1b + j * L, L)] for j in range(3 * P // L)]
            for p in range(P):
                ws = [_splat(w0vecs[p // L], p % L)]
                for k in range(3):
                    q = k * P + p
                    ws.append(_splat(w1vecs[q // L], q % L))
                for c8 in range(CV):
                    row = rows_v[row_base + p, pl.ds(c8 * L, L)]
                    for k in range(4):
                        accs[k * CV + c8] = accs[k * CV + c8] + ws[k] * row
            for c8 in range(CV):
                out0_v[v, pl.ds(c8 * L, L)] = accs[c8]
            for k in range(1, 4):
                for c8 in range(CV):
                    out1_v[(k - 1) * VB + v, pl.ds(c8 * L, L)] = \
                        accs[k * CV + c8]
            return carry

        lax.fori_loop(0, VB, v_body, 0)
        vb = vbase0 + g * VB
        pltpu.sync_copy(out0_v, out0_hbm.at[pl.ds(pl.multiple_of(vb, 8), VB)])
        for k in range(3):
            pltpu.sync_copy(
                out1_v.at[pl.ds(k * VB, VB)],
                out1_hbm.at[pl.ds(pl.multiple_of(k * N + vb, 8), VB)])

    issue_idx(0, 0)
    wait_idx(0)
    issue_rows_w(0, 0)
    issue_idx(1, 1)

    def loop_body(gg, carry):
        for b in range(2):
            g = gg * 2 + b
            wait_rows_w(b)
            wait_idx(1 - b)
            issue_rows_w(jnp.minimum(g + 1, CHUNKS - 1), 1 - b)
            issue_idx(jnp.minimum(g + 2, CHUNKS - 1), b)
            compute(g, b)
        return carry

    lax.fori_loop(0, CHUNKS // 2, loop_body, 0)
    wait_rows_w(0)
    wait_idx(1)


@jax.jit
def kernel(signal_0, kernel_0, kernel_1, patches_idx):
    table = signal_0[0, :, 0, :]
    w0f = kernel_0.transpose(0, 2, 3, 4, 1).reshape(P * N)
    w1f = kernel_1.transpose(0, 2, 3, 4, 1).reshape(P * 3 * N)
    idxf = patches_idx.transpose(0, 2, 3, 1).reshape(P * 2 * N)

    mesh = plsc.VectorSubcoreMesh(core_axis_name="c", subcore_axis_name="s")
    out0, out1 = pl.kernel(
        _sc_kernel,
        out_type=(
            jax.ShapeDtypeStruct((N, C), jnp.float32),
            jax.ShapeDtypeStruct((3 * N, C), jnp.float32),
        ),
        mesh=mesh,
        scratch_types=[
            pltpu.VMEM((NV_E, L), jnp.int32),
            pltpu.VMEM((NV_E, L), jnp.int32),
            pltpu.VMEM((NV_W1, L), jnp.int32),
            pltpu.VMEM((4, IDX_B), jnp.int32),
            pltpu.VMEM((4, IDX_B), jnp.int32),
            pltpu.VMEM((12, IDX_B), jnp.int32),
            pltpu.VMEM((4, IDX_B), jnp.int32),
            pltpu.VMEM((2 * EPC,), jnp.float32),
            pltpu.VMEM((2 * EPC * 3,), jnp.float32),
            pltpu.VMEM((2 * EPC, C), jnp.float32),
            pltpu.VMEM((VB, C), jnp.float32),
            pltpu.VMEM((3 * VB, C), jnp.float32),
            pltpu.SemaphoreType.DMA,
            pltpu.SemaphoreType.DMA,
            pltpu.SemaphoreType.DMA,
            pltpu.SemaphoreType.DMA,
        ],
    )(idxf, w0f, w1f, table)

    y0 = out0.reshape(1, N, 1, C)
    y1 = out1.reshape(3, N, C).transpose(1, 0, 2)[None]
    return (y0, y1)
